# Initial kernel scaffold; baseline (speedup 1.0000x reference)
import jax
import jax.numpy as jnp
from jax.experimental import pallas as pl


def kernel(pred, target):
    raise NotImplementedError("write your pallas kernel here")


# speedup vs baseline: 2.0776x; 2.0776x over previous
import functools
import jax
import jax.numpy as jnp
from jax.experimental import pallas as pl
from jax.experimental.pallas import tpu as pltpu

SMOOTHING = 0.1
IGNORE_INDEX = 0


def _loss_body(pred_ref, tgt_ref, out_ref, m_ref, s_ref, psum_ref, tval_ref,
               p0_ref, *, n_col_blocks, blk_cols, n_classes):
    cb = pl.program_id(0)
    x = pred_ref[...]
    rows = x.shape[0]

    col = jax.lax.broadcasted_iota(jnp.int32, (1, blk_cols), 1) + cb * blk_cols
    valid = col < n_classes

    xm = jnp.where(valid, x, -jnp.inf)
    bm = jnp.max(xm, axis=1, keepdims=True)
    xz = jnp.where(valid, x, 0.0)

    tgt = tgt_ref[...]
    hit = col == tgt
    tpart = jnp.sum(jnp.where(hit, x, 0.0), axis=1, keepdims=True)

    @pl.when(cb == 0)
    def _init():
        m_ref[...] = bm
        s_ref[...] = jnp.sum(jnp.exp(xm - bm), axis=1, keepdims=True)
        psum_ref[...] = jnp.sum(xz, axis=1, keepdims=True)
        tval_ref[...] = tpart
        p0_ref[...] = x[:, 0:1]

    @pl.when(cb != 0)
    def _acc():
        m_old = m_ref[...]
        m_new = jnp.maximum(m_old, bm)
        s_ref[...] = (s_ref[...] * jnp.exp(m_old - m_new)
                      + jnp.sum(jnp.exp(xm - m_new), axis=1, keepdims=True))
        m_ref[...] = m_new
        psum_ref[...] += jnp.sum(xz, axis=1, keepdims=True)
        tval_ref[...] += tpart

    @pl.when(cb == n_col_blocks - 1)
    def _fin():
        eps = SMOOTHING / (n_classes - 1)
        conf = 1.0 - SMOOTHING
        lse = m_ref[...] + jnp.log(s_ref[...])
        s_logp = psum_ref[...] - n_classes * lse
        tlp = tval_ref[...] - lse
        zlp = p0_ref[...] - lse
        loss = -(eps * (s_logp - tlp - zlp) + conf * tlp)
        loss = jnp.where(tgt == IGNORE_INDEX, 0.0, loss)
        out_ref[...] = jnp.sum(loss, axis=0, keepdims=True) / rows


def kernel(pred, target):
    n, c = pred.shape
    blk_cols = 2048
    n_col_blocks = pl.cdiv(c, blk_cols)
    tgt2d = target.reshape(n, 1).astype(jnp.int32)

    out = pl.pallas_call(
        functools.partial(_loss_body, n_col_blocks=n_col_blocks,
                          blk_cols=blk_cols, n_classes=c),
        grid=(n_col_blocks,),
        in_specs=[
            pl.BlockSpec((n, blk_cols), lambda cb: (0, cb)),
            pl.BlockSpec((n, 1), lambda cb: (0, 0)),
        ],
        out_specs=pl.BlockSpec((1, 1), lambda cb: (0, 0)),
        out_shape=jax.ShapeDtypeStruct((1, 1), jnp.float32),
        scratch_shapes=[
            pltpu.VMEM((n, 1), jnp.float32),
            pltpu.VMEM((n, 1), jnp.float32),
            pltpu.VMEM((n, 1), jnp.float32),
            pltpu.VMEM((n, 1), jnp.float32),
            pltpu.VMEM((n, 1), jnp.float32),
        ],
    )(pred, tgt2d)
    return out[0, 0]
